# baseline (speedup 1.0000x reference)
import functools

import jax
import jax.numpy as jnp
from jax import lax
from jax.experimental import pallas as pl
from jax.experimental.pallas import tpu as pltpu
from jax.experimental.pallas import tpu_sc as plsc

B, L, V, D, H, C = 4096, 50, 100000, 128, 300, 1000

NC, NS = 2, 16
NW = NC * NS
BW = B // NW
SPC = 2
CHUNK = SPC * L
NCHUNK = BW // SPC
NREG = D // 16


NBUF = 4


def _sc_body(idx_hbm, table_hbm, out_hbm, idx_v, rows_v, sums_v,
             sem0, sem1, sem2, sem3):
    wid = lax.axis_index("s") * NC + lax.axis_index("c")
    pltpu.sync_copy(idx_hbm.at[wid], idx_v)
    sems = (sem0, sem1, sem2, sem3)

    def copy(c, buf):
        return pltpu.make_async_copy(
            table_hbm.at[idx_v.at[c]], rows_v.at[buf], sems[buf])

    UNROLL = 5

    def accum_sample(buf, c, s_local):
        base = s_local * L

        def body(r, acc):
            row = base + r * UNROLL
            for dr in range(UNROLL):
                acc = tuple(acc[j] + rows_v[buf, row + dr, pl.ds(16 * j, 16)]
                            for j in range(NREG))
            return acc

        acc = lax.fori_loop(
            0, L // UNROLL, body,
            tuple(jnp.zeros((16,), jnp.float32) for _ in range(NREG)))
        sample = c * SPC + s_local
        for j in range(NREG):
            sums_v[sample, pl.ds(16 * j, 16)] = acc[j]

    for b in range(NBUF - 1):
        copy(b, b).start()

    def group_body(i, _):
        for buf in range(NBUF):
            c = NBUF * i + buf
            copy(c, buf).wait()

            @pl.when(c + NBUF - 1 < NCHUNK)
            def _():
                copy(c + NBUF - 1, (buf + NBUF - 1) % NBUF).start()

            for s in range(SPC):
                accum_sample(buf, c, s)
        return 0

    lax.fori_loop(0, NCHUNK // NBUF, group_body, 0)
    pltpu.sync_copy(sums_v, out_hbm.at[pl.ds(wid * BW, BW)])


@jax.jit
def _sc_gather_sums(idx3, table):
    mesh = plsc.VectorSubcoreMesh(core_axis_name="c", subcore_axis_name="s")
    return pl.kernel(
        _sc_body,
        out_type=jax.ShapeDtypeStruct((B, D), jnp.float32),
        mesh=mesh,
        scratch_types=[
            pltpu.VMEM((NCHUNK, CHUNK), jnp.int32),
            pltpu.VMEM((NBUF, CHUNK, D), jnp.float32),
            pltpu.VMEM((BW, D), jnp.float32),
        ] + [pltpu.SemaphoreType.DMA] * NBUF,
    )(idx3, table)


def _mlp_body(inp_ref, sums_ref, W1_ref, b1_ref, W2_ref, b2_ref, out_ref):
    cnt = jnp.sum((inp_ref[...] != 0).astype(jnp.float32), axis=1,
                  keepdims=True)
    avg = sums_ref[...] / jnp.maximum(cnt, 1.0)
    hT = lax.dot_general(W1_ref[...], avg, (((1,), (1,)), ((), ())),
                         preferred_element_type=jnp.float32) + b1_ref[...]
    hT = jnp.maximum(hT, 0.0)
    out_ref[...] = lax.dot_general(W2_ref[...], hT, (((1,), (0,)), ((), ())),
                                   preferred_element_type=jnp.float32
                                   ) + b2_ref[...]


@jax.jit
def _mlp(inputs, sums, W1, b1, W2, b2):
    Bm = 1024
    grid = (B // Bm,)
    return pl.pallas_call(
        _mlp_body,
        grid=grid,
        in_specs=[
            pl.BlockSpec((Bm, L), lambda i: (i, 0)),
            pl.BlockSpec((Bm, D), lambda i: (i, 0)),
            pl.BlockSpec((H, D), lambda i: (0, 0)),
            pl.BlockSpec((H, 1), lambda i: (0, 0)),
            pl.BlockSpec((C, H), lambda i: (0, 0)),
            pl.BlockSpec((C, 1), lambda i: (0, 0)),
        ],
        out_specs=pl.BlockSpec((C, Bm), lambda i: (0, i)),
        out_shape=jax.ShapeDtypeStruct((C, B), jnp.float32),
    )(inputs, sums, W1, b1, W2, b2)


def kernel(inputs, table, W1, b1, W2, b2):
    idx3 = inputs.reshape(NW, NCHUNK, CHUNK)
    sums = _sc_gather_sums(idx3, table)
    outT = _mlp(inputs, sums, W1, b1.reshape(H, 1), W2, b2.reshape(C, 1))
    return outT.T


# speedup vs baseline: 1.0192x; 1.0192x over previous
import functools

import jax
import jax.numpy as jnp
from jax import lax
from jax.experimental import pallas as pl
from jax.experimental.pallas import tpu as pltpu
from jax.experimental.pallas import tpu_sc as plsc

B, L, V, D, H, C = 4096, 50, 100000, 128, 300, 1000

NC, NS = 2, 16
NW = NC * NS
BW = B // NW
SPC = 2
CHUNK = SPC * L
NCHUNK = BW // SPC
NREG = D // 16


NBUF = 4


def _sc_body(idx_hbm, table_hbm, out_hbm, idx_v, rows_v, sums_v,
             sem0, sem1, sem2, sem3):
    wid = lax.axis_index("s") * NC + lax.axis_index("c")
    pltpu.sync_copy(idx_hbm.at[wid], idx_v)
    sems = (sem0, sem1, sem2, sem3)

    def copy(c, buf):
        return pltpu.make_async_copy(
            table_hbm.at[idx_v.at[c]], rows_v.at[buf], sems[buf])

    def accum_sample(buf, c, s_local):
        base = s_local * L

        def body(r, acc):
            return tuple(acc[j] + rows_v[buf, base + r, pl.ds(16 * j, 16)]
                         for j in range(NREG))

        acc = lax.fori_loop(
            0, L, body,
            tuple(jnp.zeros((16,), jnp.float32) for _ in range(NREG)))
        sample = c * SPC + s_local
        for j in range(NREG):
            sums_v[sample, pl.ds(16 * j, 16)] = acc[j]

    for b in range(NBUF - 1):
        copy(b, b).start()

    def group_body(i, _):
        for buf in range(NBUF):
            c = NBUF * i + buf
            copy(c, buf).wait()

            @pl.when(c + NBUF - 1 < NCHUNK)
            def _():
                copy(c + NBUF - 1, (buf + NBUF - 1) % NBUF).start()

            for s in range(SPC):
                accum_sample(buf, c, s)
        return 0

    lax.fori_loop(0, NCHUNK // NBUF, group_body, 0)
    pltpu.sync_copy(sums_v, out_hbm.at[pl.ds(wid * BW, BW)])


@jax.jit
def _sc_gather_sums(idx3, table):
    mesh = plsc.VectorSubcoreMesh(core_axis_name="c", subcore_axis_name="s")
    return pl.kernel(
        _sc_body,
        out_type=jax.ShapeDtypeStruct((B, D), jnp.float32),
        mesh=mesh,
        scratch_types=[
            pltpu.VMEM((NCHUNK, CHUNK), jnp.int32),
            pltpu.VMEM((NBUF, CHUNK, D), jnp.float32),
            pltpu.VMEM((BW, D), jnp.float32),
        ] + [pltpu.SemaphoreType.DMA] * NBUF,
    )(idx3, table)


def _mlp_body(inp_ref, sums_ref, W1_ref, b1_ref, W2_ref, b2_ref, out_ref):
    cnt = jnp.sum((inp_ref[...] != 0).astype(jnp.float32), axis=1,
                  keepdims=True)
    avg = sums_ref[...] / jnp.maximum(cnt, 1.0)
    hT = lax.dot_general(W1_ref[...], avg, (((1,), (1,)), ((), ())),
                         preferred_element_type=jnp.float32) + b1_ref[...]
    hT = jnp.maximum(hT, 0.0)
    out_ref[...] = lax.dot_general(W2_ref[...], hT, (((1,), (0,)), ((), ())),
                                   preferred_element_type=jnp.float32
                                   ) + b2_ref[...]


@jax.jit
def _mlp(inputs, sums, W1, b1, W2, b2):
    Bm = 2048
    grid = (B // Bm,)
    return pl.pallas_call(
        _mlp_body,
        grid=grid,
        in_specs=[
            pl.BlockSpec((Bm, L), lambda i: (i, 0)),
            pl.BlockSpec((Bm, D), lambda i: (i, 0)),
            pl.BlockSpec((H, D), lambda i: (0, 0)),
            pl.BlockSpec((H, 1), lambda i: (0, 0)),
            pl.BlockSpec((C, H), lambda i: (0, 0)),
            pl.BlockSpec((C, 1), lambda i: (0, 0)),
        ],
        out_specs=pl.BlockSpec((C, Bm), lambda i: (0, i)),
        out_shape=jax.ShapeDtypeStruct((C, B), jnp.float32),
    )(inputs, sums, W1, b1, W2, b2)


def kernel(inputs, table, W1, b1, W2, b2):
    idx3 = inputs.reshape(NW, NCHUNK, CHUNK)
    sums = _sc_gather_sums(idx3, table)
    outT = _mlp(inputs, sums, W1, b1.reshape(H, 1), W2, b2.reshape(C, 1))
    return outT.T
